# baseline (speedup 1.0000x reference)
import functools

import jax
import jax.numpy as jnp
import numpy as np
from jax.experimental import pallas as pl
from jax.experimental.pallas import tpu as pltpu

_F32 = jnp.float32


def _pass_a(x_ref, h_ref, w1_ref, de_ref, out_ref, *, nb):
    i = pl.program_id(0)
    xw = jnp.dot(x_ref[...], w1_ref[...], preferred_element_type=_F32)
    part = jax.lax.dot_general(h_ref[...], xw, (((0,), (0,)), ((), ())),
                               preferred_element_type=_F32)

    @pl.when(i == 0)
    def _():
        out_ref[...] = part

    @pl.when(i > 0)
    def _():
        out_ref[...] += part

    @pl.when(i == nb - 1)
    def _():
        out_ref[...] *= de_ref[...]


def _pass_b(h_ref, m_ref, emsg_ref, k_ref, wa_ref, we_ref, w2_ref, dv_ref,
            out_ref, *, nb, inv_sqrt_d):
    i = pl.program_id(0)
    x1 = jnp.maximum(
        jnp.dot(h_ref[...], emsg_ref[...], preferred_element_type=_F32)
        * dv_ref[...], 0.0)
    x1w = jnp.dot(x1, wa_ref[...], preferred_element_type=_F32)
    logits = jax.lax.dot_general(
        x1w, k_ref[...], (((1,), (1,)), ((), ())),
        preferred_element_type=_F32) * inv_sqrt_d
    mx = jnp.max(logits, axis=1, keepdims=True)
    p = jnp.exp(logits - mx)
    alpha = p / jnp.sum(p, axis=1, keepdims=True)
    am = alpha * m_ref[...]
    part = jax.lax.dot_general(am, x1, (((0,), (0,)), ((), ())),
                               preferred_element_type=_F32)

    @pl.when(i == 0)
    def _():
        out_ref[...] = part + jnp.dot(k_ref[...], we_ref[...],
                                      preferred_element_type=_F32)

    @pl.when(i > 0)
    def _():
        out_ref[...] += part

    @pl.when(i == nb - 1)
    def _():
        out_ref[...] = jnp.dot(out_ref[...], w2_ref[...],
                               preferred_element_type=_F32)


def _pass_c(m_ref, ew_ref, wp_ref, ei_ref, ej_ref, out_ref, x2_scr, *,
            nb, bn, n):
    i = pl.program_id(0)
    x2 = jnp.maximum(jnp.dot(m_ref[...], ew_ref[...],
                             preferred_element_type=_F32), 0.0)
    x2_scr[pl.ds(i * bn, bn), :] = x2

    @pl.when(i == nb - 1)
    def _():
        x2f = x2_scr[...]
        ctx = jnp.sum(x2f, axis=0, keepdims=True) * (1.0 / n)
        wc = jax.lax.dot_general(wp_ref[...], ctx, (((1,), (1,)), ((), ())),
                                 preferred_element_type=_F32)
        s = jax.lax.dot_general(wc, x2f, (((0,), (1,)), ((), ())),
                                preferred_element_type=_F32)

        def softmax_row(t):
            mx = jnp.max(t, axis=1, keepdims=True)
            p = jnp.exp(t - mx)
            return p / jnp.sum(p, axis=1, keepdims=True)

        w = softmax_row(s * ei_ref[...]) + softmax_row(s * ej_ref[...])
        out_ref[...] = jax.lax.dot_general(
            w, x2f, (((1,), (0,)), ((), ())), preferred_element_type=_F32)


def _run(x, H, K, M, Dv, De, Ei, Ej, W1, Wa, We, W2, Wp, *, interpret=False):
    n, d = x.shape
    e = K.shape[0]
    bn = 1000 if n % 1000 == 0 else 8 * (n // 8)
    nb = n // bn

    cp = pltpu.CompilerParams(dimension_semantics=("arbitrary",))
    full = lambda shape: pl.BlockSpec(shape, lambda i: (0, 0))
    rows = lambda shape: pl.BlockSpec(shape, lambda i: (i, 0))

    emsg = pl.pallas_call(
        functools.partial(_pass_a, nb=nb),
        grid=(nb,),
        in_specs=[rows((bn, d)), rows((bn, e)), full((d, d)), full((e, 1))],
        out_specs=full((e, d)),
        out_shape=jax.ShapeDtypeStruct((e, d), _F32),
        compiler_params=cp, interpret=interpret,
    )(x, H, W1, De.reshape(e, 1))

    ew = pl.pallas_call(
        functools.partial(_pass_b, nb=nb, inv_sqrt_d=float(1.0 / np.sqrt(d))),
        grid=(nb,),
        in_specs=[rows((bn, e)), rows((bn, e)), full((e, d)), full((e, d)),
                  full((d, d)), full((d, d)), full((d, d)), rows((bn, 1))],
        out_specs=full((e, d)),
        out_shape=jax.ShapeDtypeStruct((e, d), _F32),
        compiler_params=cp, interpret=interpret,
    )(H, M, emsg, K, Wa, We, W2, Dv.reshape(n, 1))

    out = pl.pallas_call(
        functools.partial(_pass_c, nb=nb, bn=bn, n=float(n)),
        grid=(nb,),
        in_specs=[rows((bn, e)), full((e, d)), full((d, d)),
                  full((1, n)), full((1, n))],
        out_specs=full((1, d)),
        out_shape=jax.ShapeDtypeStruct((1, d), _F32),
        scratch_shapes=[pltpu.VMEM((n, d), _F32)],
        compiler_params=cp, interpret=interpret,
    )(M, ew, Wp, Ei.reshape(1, n), Ej.reshape(1, n))

    return out.reshape(d)


def kernel(x, H, K, M, D_v_inv, D_e_inv, E_intra, E_inter, W1, Wa, We, W2, Wp):
    return _run(x, H, K, M, D_v_inv, D_e_inv, E_intra, E_inter,
                W1, Wa, We, W2, Wp)
